# Initial kernel scaffold; baseline (speedup 1.0000x reference)
import jax
import jax.numpy as jnp
from jax.experimental import pallas as pl


def kernel(x, T0, T1, T2, T3):
    raise NotImplementedError("write your pallas kernel here")


# speedup vs baseline: 16.8882x; 16.8882x over previous
import functools

import jax
import jax.numpy as jnp
from jax.experimental import pallas as pl
from jax.experimental.pallas import tpu as pltpu
from jax.experimental.pallas import tpu_sc as plsc

EMB = 32
N_FIELDS = 4
WINDOW = 128


def kernel(x, T0, T1, T2, T3):
    F, B, S = x.shape
    N = B * S
    xf = x.reshape(F, 1, N).astype(jnp.int32)

    mesh = plsc.VectorSubcoreMesh(
        core_axis_name="core", subcore_axis_name="subcore"
    )

    @functools.partial(
        pl.kernel,
        out_type=jax.ShapeDtypeStruct((N, N_FIELDS * EMB), jnp.float32),
        mesh=mesh,
        compiler_params=pltpu.CompilerParams(use_tc_tiling_on_sc=False),
    )
    def sc_gather(x_hbm, t0, t1, t2, t3, out_hbm):
        tables = [t0, t1, t2, t3]
        for f in range(N_FIELDS):
            table = tables[f]

            def body(i_vmem, o_vmem, table=table):
                pltpu.sync_copy(table.at[i_vmem.at[0, 0]], o_vmem)

            pltpu.emit_pipeline(
                body,
                grid=(N // WINDOW,),
                in_specs=[
                    pl.BlockSpec((1, 1, WINDOW), index_map=lambda j, f=f: (f, 0, j))
                ],
                out_specs=[
                    pl.BlockSpec((WINDOW, EMB), index_map=lambda j, f=f: (j, f))
                ],
                core_axis_name=("core", "subcore"),
                dimension_semantics=(pltpu.PARALLEL,),
            )(x_hbm, out_hbm)

    out = sc_gather(xf, T0, T1, T2, T3)
    return out.reshape(B, S, N_FIELDS * EMB)


# speedup vs baseline: 22.5161x; 1.3332x over previous
import functools

import jax
import jax.numpy as jnp
from jax.experimental import pallas as pl
from jax.experimental.pallas import tpu as pltpu
from jax.experimental.pallas import tpu_sc as plsc

EMB = 32
N_FIELDS = 4
WINDOW = 512


def kernel(x, T0, T1, T2, T3):
    F, B, S = x.shape
    N = B * S
    xf = x.reshape(F, 1, N).astype(jnp.int32)

    mesh = plsc.VectorSubcoreMesh(
        core_axis_name="core", subcore_axis_name="subcore"
    )

    @functools.partial(
        pl.kernel,
        out_type=jax.ShapeDtypeStruct((N, N_FIELDS * EMB), jnp.float32),
        mesh=mesh,
        compiler_params=pltpu.CompilerParams(use_tc_tiling_on_sc=False),
    )
    def sc_gather(x_hbm, t0, t1, t2, t3, out_hbm):
        tables = [t0, t1, t2, t3]
        for f in range(N_FIELDS):
            table = tables[f]

            def body(i_vmem, o_vmem, table=table):
                pltpu.sync_copy(table.at[i_vmem.at[0, 0]], o_vmem)

            pltpu.emit_pipeline(
                body,
                grid=(N // WINDOW,),
                in_specs=[
                    pl.BlockSpec((1, 1, WINDOW), index_map=lambda j, f=f: (f, 0, j))
                ],
                out_specs=[
                    pl.BlockSpec((WINDOW, EMB), index_map=lambda j, f=f: (j, f))
                ],
                core_axis_name=("core", "subcore"),
                dimension_semantics=(pltpu.PARALLEL,),
            )(x_hbm, out_hbm)

    out = sc_gather(xf, T0, T1, T2, T3)
    return out.reshape(B, S, N_FIELDS * EMB)


# speedup vs baseline: 25.7458x; 1.1434x over previous
import functools

import jax
import jax.numpy as jnp
from jax.experimental import pallas as pl
from jax.experimental.pallas import tpu as pltpu
from jax.experimental.pallas import tpu_sc as plsc

EMB = 32
N_FIELDS = 4
WINDOW = 1024


def kernel(x, T0, T1, T2, T3):
    F, B, S = x.shape
    N = B * S
    xf = x.reshape(F, 1, N).astype(jnp.int32)

    mesh = plsc.VectorSubcoreMesh(
        core_axis_name="core", subcore_axis_name="subcore"
    )

    @functools.partial(
        pl.kernel,
        out_type=jax.ShapeDtypeStruct((N, N_FIELDS * EMB), jnp.float32),
        mesh=mesh,
        compiler_params=pltpu.CompilerParams(use_tc_tiling_on_sc=False),
    )
    def sc_gather(x_hbm, t0, t1, t2, t3, out_hbm):
        tables = [t0, t1, t2, t3]
        for f in range(N_FIELDS):
            table = tables[f]

            def body(i_vmem, o_vmem, table=table):
                pltpu.sync_copy(table.at[i_vmem.at[0, 0]], o_vmem)

            pltpu.emit_pipeline(
                body,
                grid=(N // WINDOW,),
                in_specs=[
                    pl.BlockSpec((1, 1, WINDOW), index_map=lambda j, f=f: (f, 0, j))
                ],
                out_specs=[
                    pl.BlockSpec((WINDOW, EMB), index_map=lambda j, f=f: (j, f))
                ],
                core_axis_name=("core", "subcore"),
                dimension_semantics=(pltpu.PARALLEL,),
            )(x_hbm, out_hbm)

    out = sc_gather(xf, T0, T1, T2, T3)
    return out.reshape(B, S, N_FIELDS * EMB)


# speedup vs baseline: 25.8209x; 1.0029x over previous
import functools

import jax
import jax.numpy as jnp
from jax.experimental import pallas as pl
from jax.experimental.pallas import tpu as pltpu
from jax.experimental.pallas import tpu_sc as plsc

EMB = 32
N_FIELDS = 4
WINDOW = 1600


def kernel(x, T0, T1, T2, T3):
    F, B, S = x.shape
    N = B * S
    xf = x.reshape(F, 1, N).astype(jnp.int32)

    mesh = plsc.VectorSubcoreMesh(
        core_axis_name="core", subcore_axis_name="subcore"
    )

    @functools.partial(
        pl.kernel,
        out_type=jax.ShapeDtypeStruct((N, N_FIELDS * EMB), jnp.float32),
        mesh=mesh,
        compiler_params=pltpu.CompilerParams(use_tc_tiling_on_sc=False),
    )
    def sc_gather(x_hbm, t0, t1, t2, t3, out_hbm):
        tables = [t0, t1, t2, t3]
        for f in range(N_FIELDS):
            table = tables[f]

            def body(i_vmem, o_vmem, table=table):
                pltpu.sync_copy(table.at[i_vmem.at[0, 0]], o_vmem)

            pltpu.emit_pipeline(
                body,
                grid=(N // WINDOW,),
                in_specs=[
                    pl.BlockSpec((1, 1, WINDOW), index_map=lambda j, f=f: (f, 0, j))
                ],
                out_specs=[
                    pl.BlockSpec((WINDOW, EMB), index_map=lambda j, f=f: (j, f))
                ],
                core_axis_name=("core", "subcore"),
                dimension_semantics=(pltpu.PARALLEL,),
            )(x_hbm, out_hbm)

    out = sc_gather(xf, T0, T1, T2, T3)
    return out.reshape(B, S, N_FIELDS * EMB)
